# baseline (speedup 1.0000x reference)
import functools

import jax
import jax.numpy as jnp
from jax import lax
from jax.experimental import pallas as pl
from jax.experimental.pallas import tpu as pltpu
from jax.experimental.pallas import tpu_sc as plsc

EMBED = 64
SUB = 16
NH, ND, NM, NO = 24, 7, 12, 2
NROWS = NH * ND * NM * NO
TPAD = 4096
B, S = 4096, 200
NC, NS = 2, 16
NW = NC * NS
NEG = 4
NBP = NW // NEG
SB_TILES = 25
TILES = 4 * SB_TILES


def _fuse_body(ht_ref, dt_ref, mt_ref, ot_ref, w_ref, b_ref, out_ref):
    i = lax.broadcasted_iota(jnp.int32, (NROWS, 1), 0)
    parts = [
        (i // (ND * NM * NO), ht_ref, NH, 0),
        ((i // (NM * NO)) % ND, dt_ref, ND, 1),
        ((i // NO) % NM, mt_ref, NM, 2),
        (i % NO, ot_ref, NO, 3),
    ]
    acc = jnp.broadcast_to(b_ref[...], (EMBED, NROWS))
    for idx, tab_ref, n, j in parts:
        w_slice = w_ref[:, j * SUB:(j + 1) * SUB]
        pt = lax.dot_general(tab_ref[...], w_slice, (((1,), (1,)), ((), ())),
                             preferred_element_type=jnp.float32,
                             precision=lax.Precision.HIGHEST)
        oh = (idx == lax.broadcasted_iota(jnp.int32, (NROWS, n), 1)).astype(jnp.float32)
        acc = acc + lax.dot_general(pt, oh, (((0,), (1,)), ((), ())),
                                    preferred_element_type=jnp.float32,
                                    precision=lax.Precision.HIGHEST)
    out_ref[...] = acc


def _fuse_tables(ht, dt, mt, ot, W, b):
    return pl.pallas_call(
        _fuse_body,
        out_shape=jax.ShapeDtypeStruct((EMBED, NROWS), jnp.float32),
    )(ht, dt, mt, ot, W, b.reshape(EMBED, 1))


@functools.cache
def _make_sc_gather():
    mesh = plsc.VectorSubcoreMesh(core_axis_name="c", subcore_axis_name="s")

    @functools.partial(
        pl.kernel,
        mesh=mesh,
        compiler_params=pltpu.CompilerParams(use_tc_tiling_on_sc=False,
                                             needs_layout_passes=False),
        out_type=jax.ShapeDtypeStruct((S, EMBED // 8, B // 128, 8, 128),
                                      jnp.float32),
        scratch_types=[
            pltpu.VMEM((8 * TPAD,), jnp.int32),
            pltpu.VMEM((2, 4, 8, 128), jnp.int32),
            pltpu.VMEM((2, 8, 16, 128), jnp.float32),
            pltpu.SemaphoreType.DMA,
            pltpu.SemaphoreType.DMA,
            pltpu.SemaphoreType.DMA,
            pltpu.SemaphoreType.DMA,
            pltpu.SemaphoreType.DMA,
        ],
    )
    def _sc_gather(h4, d4, m4, o4, tab_hbm, out5,
                   tabv, ibuf, obuf, tsem, isem0, isem1, ssem0, ssem1):
        wid = lax.axis_index("s") * NC + lax.axis_index("c")
        eg = wid % NEG
        bp = wid // NEG
        isems = (isem0, isem1)
        ssems = (ssem0, ssem1)

        tcps = [pltpu.make_async_copy(tab_hbm.at[eg * 8 + e],
                                      tabv.at[pl.ds(e * TPAD, NROWS)], tsem)
                for e in range(8)]
        for cp in tcps:
            cp.start()
        for cp in tcps:
            cp.wait()

        def tile_coords(t):
            return bp * 4 + t // SB_TILES, t % SB_TILES

        def idx_copies(t, slot):
            b0, s0 = tile_coords(t)
            sem = isems[slot]
            return [pltpu.make_async_copy(src.at[s0, b0], ibuf.at[slot, j], sem)
                    for j, src in enumerate((h4, d4, m4, o4))]

        def scatter_copies(t, slot):
            b0, s0 = tile_coords(t)
            sem = ssems[slot]
            return [pltpu.make_async_copy(
                obuf.at[slot, :, pl.ds(h * 8, 8), :],
                out5.at[pl.ds(s0 * 8, 8), eg * 2 + h, b0],
                sem) for h in range(2)]

        for cp in idx_copies(0, 0):
            cp.start()

        ebase = [jnp.full((16,), e * TPAD, jnp.int32) for e in range(8)]
        himask = jnp.full((16,), -65536, jnp.int32)

        def group_body(g, _):
            for p in range(2):
                t = g * 2 + p

                for cp in idx_copies(t, p):
                    cp.wait()

                @pl.when(t + 1 < TILES)
                def _():
                    for cp in idx_copies(t + 1, 1 - p):
                        cp.start()

                @pl.when(t >= 2)
                def _():
                    for cp in scatter_copies(t - 2, p):
                        cp.wait()

                @plsc.parallel_loop(0, 8, 1, unroll=3)
                def sr_body(sr):
                    fv = []
                    for brg in range(8):
                        sl = pl.ds(brg * 16, 16)
                        hv = ibuf[p, 0, sr, sl]
                        dv = ibuf[p, 1, sr, sl]
                        mv = ibuf[p, 2, sr, sl]
                        ov = ibuf[p, 3, sr, sl]
                        fv.append((hv * (ND * NM * NO) + dv * (NM * NO))
                                  + (mv * NO + ov))
                    for brg in range(8):
                        sl = pl.ds(brg * 16, 16)
                        for ep in range(8):
                            v = plsc.load_gather(tabv, [fv[brg] + ebase[ep]])
                            hi = plsc.bitcast(v & himask, jnp.float32)
                            lo = plsc.bitcast(v << 16, jnp.float32)
                            obuf[p, sr, 2 * ep, sl] = hi
                            obuf[p, sr, 2 * ep + 1, sl] = lo

                for cp in scatter_copies(t, p):
                    cp.start()
            return 0

        lax.fori_loop(0, TILES // 2, group_body, 0)

        for t in (TILES - 2, TILES - 1):
            for cp in scatter_copies(t, t % 2):
                cp.wait()

    return _sc_gather


def _as_tiles(x):
    return x.astype(jnp.int32).T.reshape(SB_TILES, 8, B // 128, 128).transpose(0, 2, 1, 3)


def _pack_pairs(ft):
    bf = ft.astype(jnp.bfloat16)
    hi = lax.bitcast_convert_type(bf[0::2], jnp.uint16).astype(jnp.uint32)
    lo = lax.bitcast_convert_type(bf[1::2], jnp.uint16).astype(jnp.uint32)
    return lax.bitcast_convert_type((hi << 16) | lo, jnp.int32)


def kernel(hours, days, months, holidays, hour_table, day_table, month_table,
           holiday_table, W, b):
    fused_t = _fuse_tables(hour_table, day_table, month_table, holiday_table, W, b)
    out5 = _make_sc_gather()(_as_tiles(hours), _as_tiles(days),
                             _as_tiles(months), _as_tiles(holidays),
                             _pack_pairs(fused_t))
    return out5.transpose(2, 4, 0, 1, 3).reshape(B, S, EMBED)


# speedup vs baseline: 1.8259x; 1.8259x over previous
import functools

import jax
import jax.numpy as jnp
from jax import lax
from jax.experimental import pallas as pl
from jax.experimental.pallas import tpu as pltpu
from jax.experimental.pallas import tpu_sc as plsc

EMBED = 64
SUB = 16
NH, ND, NM, NO = 24, 7, 12, 2
NROWS = NH * ND * NM * NO
TPAD = 4096
B, S = 4096, 200
NC, NS = 2, 16
NW = NC * NS
NEG = 4
NBP = NW // NEG
SB_TILES = 25
TILES = 4 * SB_TILES


def _fuse_body(ht_ref, dt_ref, mt_ref, ot_ref, w_ref, b_ref, out_ref):
    i = lax.broadcasted_iota(jnp.int32, (NROWS, 1), 0)
    parts = [
        (i // (ND * NM * NO), ht_ref, NH, 0),
        ((i // (NM * NO)) % ND, dt_ref, ND, 1),
        ((i // NO) % NM, mt_ref, NM, 2),
        (i % NO, ot_ref, NO, 3),
    ]
    acc = jnp.broadcast_to(b_ref[...], (EMBED, NROWS))
    for idx, tab_ref, n, j in parts:
        w_slice = w_ref[:, j * SUB:(j + 1) * SUB]
        pt = lax.dot_general(tab_ref[...], w_slice, (((1,), (1,)), ((), ())),
                             preferred_element_type=jnp.float32,
                             precision=lax.Precision.HIGHEST)
        oh = (idx == lax.broadcasted_iota(jnp.int32, (NROWS, n), 1)).astype(jnp.float32)
        acc = acc + lax.dot_general(pt, oh, (((0,), (1,)), ((), ())),
                                    preferred_element_type=jnp.float32,
                                    precision=lax.Precision.HIGHEST)
    out_ref[...] = acc


def _fuse_tables(ht, dt, mt, ot, W, b):
    return pl.pallas_call(
        _fuse_body,
        out_shape=jax.ShapeDtypeStruct((EMBED, NROWS), jnp.float32),
    )(ht, dt, mt, ot, W, b.reshape(EMBED, 1))


@functools.cache
def _make_sc_gather():
    mesh = plsc.VectorSubcoreMesh(core_axis_name="c", subcore_axis_name="s")

    @functools.partial(
        pl.kernel,
        mesh=mesh,
        compiler_params=pltpu.CompilerParams(use_tc_tiling_on_sc=False,
                                             needs_layout_passes=False),
        out_type=jax.ShapeDtypeStruct((S, EMBED // 8, B // 128, 8, 128),
                                      jnp.float32),
        scratch_types=[
            pltpu.VMEM((8 * TPAD,), jnp.int32),
            pltpu.VMEM((2, 4, 8, 128), jnp.int32),
            pltpu.VMEM((2, 8, 16, 128), jnp.float32),
            pltpu.SemaphoreType.DMA,
            pltpu.SemaphoreType.DMA,
            pltpu.SemaphoreType.DMA,
            pltpu.SemaphoreType.DMA,
            pltpu.SemaphoreType.DMA,
        ],
    )
    def _sc_gather(h4, d4, m4, o4, tab_hbm, out5,
                   tabv, ibuf, obuf, tsem, isem0, isem1, ssem0, ssem1):
        wid = lax.axis_index("s") * NC + lax.axis_index("c")
        eg = wid % NEG
        bp = wid // NEG
        isems = (isem0, isem1)
        ssems = (ssem0, ssem1)

        tcps = [pltpu.make_async_copy(tab_hbm.at[eg * 8 + e],
                                      tabv.at[pl.ds(e * TPAD, NROWS)], tsem)
                for e in range(8)]
        for cp in tcps:
            cp.start()
        for cp in tcps:
            cp.wait()

        def tile_coords(t):
            return bp * 4 + t // SB_TILES, t % SB_TILES

        def idx_copies(t, slot):
            b0, s0 = tile_coords(t)
            sem = isems[slot]
            return [pltpu.make_async_copy(src.at[s0, b0], ibuf.at[slot, j], sem)
                    for j, src in enumerate((h4, d4, m4, o4))]

        def scatter_copies(t, slot):
            b0, s0 = tile_coords(t)
            sem = ssems[slot]
            return [pltpu.make_async_copy(
                obuf.at[slot, :, pl.ds(h * 8, 8), :],
                out5.at[pl.ds(s0 * 8, 8), eg * 2 + h, b0],
                sem) for h in range(2)]

        for cp in idx_copies(0, 0):
            cp.start()

        ebase = [jnp.full((16,), e * TPAD, jnp.int32) for e in range(8)]
        himask = jnp.full((16,), -65536, jnp.int32)

        def group_body(g, _):
            for p in range(2):
                t = g * 2 + p

                for cp in idx_copies(t, p):
                    cp.wait()

                @pl.when(t + 1 < TILES)
                def _():
                    for cp in idx_copies(t + 1, 1 - p):
                        cp.start()

                @pl.when(t >= 2)
                def _():
                    for cp in scatter_copies(t - 2, p):
                        cp.wait()

                @plsc.parallel_loop(0, 8, 1, unroll=2)
                def sr_body(sr):
                    fv = []
                    for brg in range(8):
                        sl = pl.ds(brg * 16, 16)
                        hv = ibuf[p, 0, sr, sl]
                        dv = ibuf[p, 1, sr, sl]
                        mv = ibuf[p, 2, sr, sl]
                        ov = ibuf[p, 3, sr, sl]
                        fv.append((hv * (ND * NM * NO) + dv * (NM * NO))
                                  + (mv * NO + ov))
                    for brg in range(8):
                        sl = pl.ds(brg * 16, 16)
                        for ep in range(8):
                            v = plsc.load_gather(tabv, [fv[brg] + ebase[ep]])
                            hi = plsc.bitcast(v, jnp.float32)
                            lo = plsc.bitcast(v << 16, jnp.float32)
                            obuf[p, sr, 2 * ep, sl] = hi
                            obuf[p, sr, 2 * ep + 1, sl] = lo

                for cp in scatter_copies(t, p):
                    cp.start()
            return 0

        lax.fori_loop(0, TILES // 2, group_body, 0)

        for t in (TILES - 2, TILES - 1):
            for cp in scatter_copies(t, t % 2):
                cp.wait()

    return _sc_gather


def _as_tiles(x):
    return x.astype(jnp.int32).T.reshape(SB_TILES, 8, B // 128, 128).transpose(0, 2, 1, 3)


def _pack_pairs(ft):
    bf = ft.astype(jnp.bfloat16)
    hi = lax.bitcast_convert_type(bf[0::2], jnp.uint16).astype(jnp.uint32)
    lo = lax.bitcast_convert_type(bf[1::2], jnp.uint16).astype(jnp.uint32)
    return lax.bitcast_convert_type((hi << 16) | lo, jnp.int32)


def kernel(hours, days, months, holidays, hour_table, day_table, month_table,
           holiday_table, W, b):
    fused_t = _fuse_tables(hour_table, day_table, month_table, holiday_table, W, b)
    out5 = _make_sc_gather()(_as_tiles(hours), _as_tiles(days),
                             _as_tiles(months), _as_tiles(holidays),
                             _pack_pairs(fused_t))
    return out5.transpose(2, 4, 0, 1, 3).reshape(B, S, EMBED)


# speedup vs baseline: 1.8289x; 1.0016x over previous
import functools

import jax
import jax.numpy as jnp
from jax import lax
from jax.experimental import pallas as pl
from jax.experimental.pallas import tpu as pltpu
from jax.experimental.pallas import tpu_sc as plsc

EMBED = 64
SUB = 16
NH, ND, NM, NO = 24, 7, 12, 2
NROWS = NH * ND * NM * NO
TPAD = 4096
B, S = 4096, 200
NC, NS = 2, 16
NW = NC * NS
NEG = 4
NBP = NW // NEG
SB_TILES = 25
TILES = 4 * SB_TILES


def _fuse_body(ht_ref, dt_ref, mt_ref, ot_ref, w_ref, b_ref, out_ref):
    i = lax.broadcasted_iota(jnp.int32, (NROWS, 1), 0)
    parts = [
        (i // (ND * NM * NO), ht_ref, NH, 0),
        ((i // (NM * NO)) % ND, dt_ref, ND, 1),
        ((i // NO) % NM, mt_ref, NM, 2),
        (i % NO, ot_ref, NO, 3),
    ]
    acc = jnp.broadcast_to(b_ref[...], (EMBED, NROWS))
    for idx, tab_ref, n, j in parts:
        w_slice = w_ref[:, j * SUB:(j + 1) * SUB]
        pt = lax.dot_general(tab_ref[...], w_slice, (((1,), (1,)), ((), ())),
                             preferred_element_type=jnp.float32,
                             precision=lax.Precision.HIGHEST)
        oh = (idx == lax.broadcasted_iota(jnp.int32, (NROWS, n), 1)).astype(jnp.float32)
        acc = acc + lax.dot_general(pt, oh, (((0,), (1,)), ((), ())),
                                    preferred_element_type=jnp.float32,
                                    precision=lax.Precision.HIGHEST)
    out_ref[...] = acc


def _fuse_tables(ht, dt, mt, ot, W, b):
    return pl.pallas_call(
        _fuse_body,
        out_shape=jax.ShapeDtypeStruct((EMBED, NROWS), jnp.float32),
    )(ht, dt, mt, ot, W, b.reshape(EMBED, 1))


@functools.cache
def _make_sc_gather():
    mesh = plsc.VectorSubcoreMesh(core_axis_name="c", subcore_axis_name="s")

    @functools.partial(
        pl.kernel,
        mesh=mesh,
        compiler_params=pltpu.CompilerParams(use_tc_tiling_on_sc=False,
                                             needs_layout_passes=False),
        out_type=jax.ShapeDtypeStruct((S, EMBED // 8, B // 128, 8, 128),
                                      jnp.float32),
        scratch_types=[
            pltpu.VMEM((8 * TPAD,), jnp.int32),
            pltpu.VMEM((2, 4, 8, 128), jnp.int32),
            pltpu.VMEM((2, 8, 16, 128), jnp.float32),
            pltpu.SemaphoreType.DMA,
            pltpu.SemaphoreType.DMA,
            pltpu.SemaphoreType.DMA,
            pltpu.SemaphoreType.DMA,
            pltpu.SemaphoreType.DMA,
        ],
    )
    def _sc_gather(h4, d4, m4, o4, tab_hbm, out5,
                   tabv, ibuf, obuf, tsem, isem0, isem1, ssem0, ssem1):
        wid = lax.axis_index("s") * NC + lax.axis_index("c")
        eg = wid % NEG
        bp = wid // NEG
        isems = (isem0, isem1)
        ssems = (ssem0, ssem1)

        tcps = [pltpu.make_async_copy(tab_hbm.at[eg * 8 + e],
                                      tabv.at[pl.ds(e * TPAD, NROWS)], tsem)
                for e in range(8)]
        for cp in tcps:
            cp.start()
        for cp in tcps:
            cp.wait()

        def tile_coords(t):
            return bp * 4 + t // SB_TILES, t % SB_TILES

        def idx_copies(t, slot):
            b0, s0 = tile_coords(t)
            sem = isems[slot]
            return [pltpu.make_async_copy(src.at[s0, b0], ibuf.at[slot, j], sem)
                    for j, src in enumerate((h4, d4, m4, o4))]

        def scatter_copies(t, slot):
            b0, s0 = tile_coords(t)
            sem = ssems[slot]
            return [pltpu.make_async_copy(
                obuf.at[slot, :, pl.ds(h * 8, 8), :],
                out5.at[pl.ds(s0 * 8, 8), eg * 2 + h, b0],
                sem) for h in range(2)]

        for cp in idx_copies(0, 0):
            cp.start()

        ebase = [jnp.full((16,), e * TPAD, jnp.int32) for e in range(8)]

        def group_body(g, _):
            for p in range(2):
                t = g * 2 + p

                for cp in idx_copies(t, p):
                    cp.wait()

                @pl.when(t + 1 < TILES)
                def _():
                    for cp in idx_copies(t + 1, 1 - p):
                        cp.start()

                @pl.when(t >= 2)
                def _():
                    for cp in scatter_copies(t - 2, p):
                        cp.wait()

                @plsc.parallel_loop(0, 8, 1, unroll=2)
                def sr_body(sr):
                    fv = []
                    for brg in range(8):
                        sl = pl.ds(brg * 16, 16)
                        hv = ibuf[p, 0, sr, sl]
                        dv = ibuf[p, 1, sr, sl]
                        mv = ibuf[p, 2, sr, sl]
                        ov = ibuf[p, 3, sr, sl]
                        fv.append((hv * (ND * NM * NO) + dv * (NM * NO))
                                  + (mv * NO + ov))
                    for brg in range(8):
                        sl = pl.ds(brg * 16, 16)
                        for ep in range(8):
                            v = plsc.load_gather(tabv, [fv[brg] + ebase[ep]])
                            hi = plsc.bitcast(v, jnp.float32)
                            lo = plsc.bitcast(v << 16, jnp.float32)
                            obuf[p, sr, 2 * ep, sl] = hi
                            obuf[p, sr, 2 * ep + 1, sl] = lo

                for cp in scatter_copies(t, p):
                    cp.start()
            return 0

        lax.fori_loop(0, TILES // 2, group_body, 0)

        for t in (TILES - 2, TILES - 1):
            for cp in scatter_copies(t, t % 2):
                cp.wait()

    return _sc_gather


def _as_tiles(x):
    return x.astype(jnp.int32).T.reshape(SB_TILES, 8, B // 128, 128).transpose(0, 2, 1, 3)


def _pack_pairs(ft):
    bf = ft.astype(jnp.bfloat16)
    hi = lax.bitcast_convert_type(bf[0::2], jnp.uint16).astype(jnp.uint32)
    lo = lax.bitcast_convert_type(bf[1::2], jnp.uint16).astype(jnp.uint32)
    return lax.bitcast_convert_type((hi << 16) | lo, jnp.int32)


def kernel(hours, days, months, holidays, hour_table, day_table, month_table,
           holiday_table, W, b):
    fused_t = _fuse_tables(hour_table, day_table, month_table, holiday_table, W, b)
    out5 = _make_sc_gather()(_as_tiles(hours), _as_tiles(days),
                             _as_tiles(months), _as_tiles(holidays),
                             _pack_pairs(fused_t))
    return out5.transpose(2, 4, 0, 1, 3).reshape(B, S, EMBED)
